# Initial kernel scaffold; baseline (speedup 1.0000x reference)
import jax
import jax.numpy as jnp
from jax.experimental import pallas as pl


def kernel(seq_emb, x, edge_index, batch, W1, b1, W2, b2, Wf, bf, Wtp, btp, Wp, bp):
    raise NotImplementedError("write your pallas kernel here")


# speedup vs baseline: 8.2011x; 8.2011x over previous
import functools

import jax
import jax.numpy as jnp
from jax import lax
from jax.experimental import pallas as pl
from jax.experimental.pallas import tpu as pltpu
from jax.experimental.pallas import tpu_sc as plsc

N = 10000
E = 320000
D = 128
H = 128
B = 256

NC = 2
NS = 16
NW = NC * NS
CHUNK = 128
CPT = 80
ET = NW * CPT * CHUNK
NPAD = 10112
ZR = NPAD // NS
HW = D // 2
DEGW = 16


def _row_chunks(total, step):
    off = 0
    while off < total:
        yield off, min(step, total - off)
        off += step


@functools.cache
def _sc_kernels():
    mesh = plsc.VectorSubcoreMesh(core_axis_name="c", subcore_axis_name="s",
                                  num_cores=NC, num_subcores=NS)

    @functools.partial(
        pl.kernel,
        out_type=jax.ShapeDtypeStruct((NC, NPAD, DEGW), jnp.float32),
        mesh=mesh,
        compiler_params=pltpu.CompilerParams(use_tc_tiling_on_sc=False),
        scratch_types=[
            pltpu.VMEM((CPT, CHUNK), jnp.int32),
            pltpu.VMEM((CHUNK, DEGW), jnp.float32),
            pltpu.VMEM((ZR, DEGW), jnp.float32),
            pltpu.VMEM_SHARED((NPAD, DEGW), jnp.float32),
        ],
    )
    def _sc_degree(dst_hbm, ones_hbm, zeros_hbm, out_hbm, dst_v, ones_v, zb, deg_sp):
        c = lax.axis_index("c")
        s = lax.axis_index("s")
        wid = c * NS + s
        pltpu.sync_copy(dst_hbm.at[wid], dst_v)
        pltpu.sync_copy(ones_hbm, ones_v)
        pltpu.sync_copy(zeros_hbm, zb)
        pltpu.sync_copy(zb, deg_sp.at[pl.ds(s * ZR, ZR)])
        plsc.subcore_barrier()

        def step(j, carry):
            pltpu.sync_copy(ones_v, deg_sp.at[dst_v.at[j]], add=True)
            return carry

        lax.fori_loop(0, CPT, step, 0)
        plsc.subcore_barrier()
        pltpu.sync_copy(deg_sp.at[pl.ds(s * ZR, ZR)], zb)
        pltpu.sync_copy(zb, out_hbm.at[c].at[pl.ds(s * ZR, ZR)])

    @functools.partial(
        pl.kernel,
        out_type=[jax.ShapeDtypeStruct((NC, NPAD, HW), jnp.float32),
                  jax.ShapeDtypeStruct((NC, NPAD, HW), jnp.float32)],
        mesh=mesh,
        compiler_params=pltpu.CompilerParams(use_tc_tiling_on_sc=False),
        scratch_types=[
            pltpu.VMEM((CPT, CHUNK), jnp.int32),
            pltpu.VMEM((CPT, CHUNK), jnp.int32),
            pltpu.VMEM((CHUNK, HW), jnp.float32),
            pltpu.VMEM((CHUNK, HW), jnp.float32),
            pltpu.VMEM_SHARED((NPAD, HW), jnp.float32),
            pltpu.SemaphoreType.DMA,
            pltpu.SemaphoreType.DMA,
        ],
    )
    def _sc_msgpass(ya_hbm, yb_hbm, src_hbm, dst_hbm, zeros_hbm, outa_hbm,
                    outb_hbm, src_v, dst_v, g0, g1, acc_sp, sem0, sem1):
        c = lax.axis_index("c")
        s = lax.axis_index("s")
        wid = c * NS + s
        pltpu.sync_copy(src_hbm.at[wid], src_v)
        pltpu.sync_copy(dst_hbm.at[wid], dst_v)

        for y_hbm, out_hbm in ((ya_hbm, outa_hbm), (yb_hbm, outb_hbm)):
            pltpu.sync_copy(zeros_hbm, g0)
            for off, sz in _row_chunks(ZR, CHUNK):
                pltpu.sync_copy(g0.at[pl.ds(0, sz)],
                                acc_sp.at[pl.ds(s * ZR + off, sz)])
            plsc.subcore_barrier()

            def step(j, carry):
                j0 = 2 * j
                j1 = 2 * j + 1
                d0 = pltpu.async_copy(y_hbm.at[src_v.at[j0]], g0, sem0)
                d1 = pltpu.async_copy(y_hbm.at[src_v.at[j1]], g1, sem1)
                d0.wait()
                pltpu.sync_copy(g0, acc_sp.at[dst_v.at[j0]], add=True)
                d1.wait()
                pltpu.sync_copy(g1, acc_sp.at[dst_v.at[j1]], add=True)
                return carry

            lax.fori_loop(0, CPT // 2, step, 0)
            plsc.subcore_barrier()
            for off, sz in _row_chunks(ZR, CHUNK):
                pltpu.sync_copy(acc_sp.at[pl.ds(s * ZR + off, sz)],
                                g0.at[pl.ds(0, sz)])
                pltpu.sync_copy(g0.at[pl.ds(0, sz)],
                                out_hbm.at[c].at[pl.ds(s * ZR + off, sz)])

    return _sc_degree, _sc_msgpass


_RB = 2000
_GRID = N // _RB


def _tc1_body(x_ref, w1_ref, degp_ref, ya_ref, yb_ref, dinv_ref):
    deg = degp_ref[0, :, 0:1] + degp_ref[1, :, 0:1] + 1.0
    dinv = lax.rsqrt(deg)
    xw = jnp.dot(x_ref[...], w1_ref[...], preferred_element_type=jnp.float32)
    y = xw * dinv
    ya_ref[...] = y[:, :HW]
    yb_ref[...] = y[:, HW:]
    dinv_ref[...] = dinv


def _tc1(x, w1, degp):
    return pl.pallas_call(
        _tc1_body,
        grid=(_GRID,),
        in_specs=[
            pl.BlockSpec((_RB, D), lambda i: (i, 0)),
            pl.BlockSpec((D, H), lambda i: (0, 0)),
            pl.BlockSpec((NC, _RB, DEGW), lambda i: (0, i, 0)),
        ],
        out_specs=[
            pl.BlockSpec((_RB, HW), lambda i: (i, 0)),
            pl.BlockSpec((_RB, HW), lambda i: (i, 0)),
            pl.BlockSpec((_RB, 1), lambda i: (i, 0)),
        ],
        out_shape=[
            jax.ShapeDtypeStruct((N, HW), jnp.float32),
            jax.ShapeDtypeStruct((N, HW), jnp.float32),
            jax.ShapeDtypeStruct((N, 1), jnp.float32),
        ],
    )(x, w1, degp)


def _gcn_out(acca_ref, accb_ref, ya_ref, yb_ref, dinv, b_ref):
    ha = acca_ref[0] + acca_ref[1] + ya_ref[...]
    hb = accb_ref[0] + accb_ref[1] + yb_ref[...]
    h = jnp.concatenate([ha, hb], axis=1) * dinv + b_ref[...]
    return jnp.maximum(h, 0.0)


def _tc2_body(acca_ref, accb_ref, ya_ref, yb_ref, dinv_ref, b1_ref, w2_ref,
              y2a_ref, y2b_ref):
    dinv = dinv_ref[...]
    h = _gcn_out(acca_ref, accb_ref, ya_ref, yb_ref, dinv, b1_ref)
    y2 = jnp.dot(h, w2_ref[...], preferred_element_type=jnp.float32) * dinv
    y2a_ref[...] = y2[:, :HW]
    y2b_ref[...] = y2[:, HW:]


def _tc2(acca, accb, ya, yb, dinv, b1, w2):
    return pl.pallas_call(
        _tc2_body,
        grid=(_GRID,),
        in_specs=[
            pl.BlockSpec((NC, _RB, HW), lambda i: (0, i, 0)),
            pl.BlockSpec((NC, _RB, HW), lambda i: (0, i, 0)),
            pl.BlockSpec((_RB, HW), lambda i: (i, 0)),
            pl.BlockSpec((_RB, HW), lambda i: (i, 0)),
            pl.BlockSpec((_RB, 1), lambda i: (i, 0)),
            pl.BlockSpec((1, H), lambda i: (0, 0)),
            pl.BlockSpec((H, H), lambda i: (0, 0)),
        ],
        out_specs=[
            pl.BlockSpec((_RB, HW), lambda i: (i, 0)),
            pl.BlockSpec((_RB, HW), lambda i: (i, 0)),
        ],
        out_shape=[
            jax.ShapeDtypeStruct((N, HW), jnp.float32),
            jax.ShapeDtypeStruct((N, HW), jnp.float32),
        ],
    )(acca, accb, ya, yb, dinv, b1, w2)


def _tc3_body(acca_ref, accb_ref, ya_ref, yb_ref, dinv_ref, b2_ref, batch_ref,
              seq_ref, wf_ref, bf_ref, wcat_ref, bcat_ref, out_ref,
              sums_ref, cnt_ref):
    i = pl.program_id(0)
    g = _gcn_out(acca_ref, accb_ref, ya_ref, yb_ref, dinv_ref[...], b2_ref)
    ids = batch_ref[...]
    cols = lax.broadcasted_iota(jnp.int32, (_RB, B), 1)
    a = (cols == ids).astype(jnp.float32)
    pa = lax.dot_general(a, g, (((0,), (0,)), ((), ())),
                         preferred_element_type=jnp.float32)
    ca = lax.dot_general(a, jnp.ones((_RB, 1), jnp.float32),
                         (((0,), (0,)), ((), ())),
                         preferred_element_type=jnp.float32)

    @pl.when(i == 0)
    def _():
        sums_ref[...] = jnp.zeros_like(sums_ref)
        cnt_ref[...] = jnp.zeros_like(cnt_ref)

    sums_ref[...] += pa
    cnt_ref[...] += ca

    @pl.when(i == _GRID - 1)
    def _():
        pooled = sums_ref[...] / jnp.maximum(cnt_ref[...], 1.0)
        fused = jnp.concatenate([seq_ref[...], pooled], axis=1)
        f2 = jnp.maximum(
            jnp.dot(fused, wf_ref[...], preferred_element_type=jnp.float32)
            + bf_ref[...], 0.0)
        out_ref[...] = (jnp.dot(f2, wcat_ref[...],
                                preferred_element_type=jnp.float32)
                        + bcat_ref[...])


def _tc3(acca, accb, ya, yb, dinv, b2, batch2d, seq_emb, wf, bf, wcat, bcat):
    return pl.pallas_call(
        _tc3_body,
        grid=(_GRID,),
        in_specs=[
            pl.BlockSpec((NC, _RB, HW), lambda i: (0, i, 0)),
            pl.BlockSpec((NC, _RB, HW), lambda i: (0, i, 0)),
            pl.BlockSpec((_RB, HW), lambda i: (i, 0)),
            pl.BlockSpec((_RB, HW), lambda i: (i, 0)),
            pl.BlockSpec((_RB, 1), lambda i: (i, 0)),
            pl.BlockSpec((1, H), lambda i: (0, 0)),
            pl.BlockSpec((_RB, 1), lambda i: (i, 0)),
            pl.BlockSpec((B, D), lambda i: (0, 0)),
            pl.BlockSpec((D + H, 128), lambda i: (0, 0)),
            pl.BlockSpec((1, 128), lambda i: (0, 0)),
            pl.BlockSpec((128, 128), lambda i: (0, 0)),
            pl.BlockSpec((1, 128), lambda i: (0, 0)),
        ],
        out_specs=pl.BlockSpec((B, 128), lambda i: (0, 0)),
        out_shape=jax.ShapeDtypeStruct((B, 128), jnp.float32),
        scratch_shapes=[
            pltpu.VMEM((B, H), jnp.float32),
            pltpu.VMEM((B, 1), jnp.float32),
        ],
    )(acca, accb, ya, yb, dinv, b2, batch2d, seq_emb, wf, bf, wcat, bcat)


def kernel(seq_emb, x, edge_index, batch, W1, b1, W2, b2, Wf, bf, Wtp, btp, Wp, bp):
    pad = ET - E
    src_p = jnp.concatenate(
        [edge_index[0], jnp.zeros((pad,), jnp.int32)]).reshape(NW, CPT, CHUNK)
    dst_p = jnp.concatenate(
        [edge_index[1], jnp.full((pad,), N, jnp.int32)]).reshape(NW, CPT, CHUNK)

    ones_deg = jnp.ones((CHUNK, DEGW), jnp.float32)
    zeros_deg = jnp.zeros((ZR, DEGW), jnp.float32)
    zeros_row = jnp.zeros((CHUNK, HW), jnp.float32)

    sc_degree, sc_msgpass = _sc_kernels()
    degp = sc_degree(dst_p, ones_deg, zeros_deg)

    y1a, y1b, dinv = _tc1(x, W1, degp)
    acc1a, acc1b = sc_msgpass(y1a, y1b, src_p, dst_p, zeros_row)
    y2a, y2b = _tc2(acc1a, acc1b, y1a, y1b, dinv, b1.reshape(1, H), W2)
    acc2a, acc2b = sc_msgpass(y2a, y2b, src_p, dst_p, zeros_row)

    wcat = jnp.zeros((128, 128), jnp.float32)
    wcat = wcat.at[:, 0:2].set(Wtp).at[:, 2:3].set(Wp)
    bcat = jnp.zeros((1, 128), jnp.float32)
    bcat = bcat.at[0, 0:2].set(btp).at[0, 2:3].set(bp)

    out3 = _tc3(acc2a, acc2b, y2a, y2b, dinv, b2.reshape(1, H),
                batch.reshape(N, 1), seq_emb, Wf, bf.reshape(1, 128),
                wcat, bcat)
    return (out3[:, 0:2], out3[:, 2:3])


# speedup vs baseline: 8.6903x; 1.0597x over previous
import functools

import jax
import jax.numpy as jnp
from jax import lax
from jax.experimental import pallas as pl
from jax.experimental.pallas import tpu as pltpu
from jax.experimental.pallas import tpu_sc as plsc

N = 10000
E = 320000
D = 128
H = 128
B = 256

NC = 2
NS = 16
NW = NC * NS
CHUNK = 128
CPT = 80
ET = NW * CPT * CHUNK
NPAD = 10112
ZR = NPAD // NS
HW = D // 2
DEGW = 16
NSLOT = 4


def _row_chunks(total, step):
    off = 0
    while off < total:
        yield off, min(step, total - off)
        off += step


@functools.cache
def _sc_kernels():
    mesh = plsc.VectorSubcoreMesh(core_axis_name="c", subcore_axis_name="s",
                                  num_cores=NC, num_subcores=NS)

    @functools.partial(
        pl.kernel,
        out_type=jax.ShapeDtypeStruct((NC, NPAD, DEGW), jnp.float32),
        mesh=mesh,
        compiler_params=pltpu.CompilerParams(use_tc_tiling_on_sc=False),
        scratch_types=[
            pltpu.VMEM((CPT, CHUNK), jnp.int32),
            pltpu.VMEM((CHUNK, DEGW), jnp.float32),
            pltpu.VMEM((ZR, DEGW), jnp.float32),
            pltpu.VMEM_SHARED((NPAD, DEGW), jnp.float32),
        ],
    )
    def _sc_degree(dst_hbm, ones_hbm, zeros_hbm, out_hbm, dst_v, ones_v, zb, deg_sp):
        c = lax.axis_index("c")
        s = lax.axis_index("s")
        wid = c * NS + s
        pltpu.sync_copy(dst_hbm.at[wid], dst_v)
        pltpu.sync_copy(ones_hbm, ones_v)
        pltpu.sync_copy(zeros_hbm, zb)
        pltpu.sync_copy(zb, deg_sp.at[pl.ds(s * ZR, ZR)])
        plsc.subcore_barrier()

        def step(j, carry):
            pltpu.sync_copy(ones_v, deg_sp.at[dst_v.at[j]], add=True)
            return carry

        lax.fori_loop(0, CPT, step, 0)
        plsc.subcore_barrier()
        pltpu.sync_copy(deg_sp.at[pl.ds(s * ZR, ZR)], zb)
        pltpu.sync_copy(zb, out_hbm.at[c].at[pl.ds(s * ZR, ZR)])

    @functools.partial(
        pl.kernel,
        out_type=[jax.ShapeDtypeStruct((NC, NPAD, HW), jnp.float32),
                  jax.ShapeDtypeStruct((NC, NPAD, HW), jnp.float32)],
        mesh=mesh,
        compiler_params=pltpu.CompilerParams(use_tc_tiling_on_sc=False),
        scratch_types=[
            pltpu.VMEM((CPT, CHUNK), jnp.int32),
            pltpu.VMEM((CPT, CHUNK), jnp.int32),
            pltpu.VMEM((2 * NSLOT, CHUNK, HW), jnp.float32),
            pltpu.VMEM_SHARED((NPAD, HW), jnp.float32),
            pltpu.SemaphoreType.DMA,
            pltpu.SemaphoreType.DMA,
            pltpu.SemaphoreType.DMA,
            pltpu.SemaphoreType.DMA,
        ],
    )
    def _sc_msgpass(ya_hbm, yb_hbm, src_hbm, dst_hbm, zeros_hbm, outa_hbm,
                    outb_hbm, src_v, dst_v, gbuf, acc_sp, gsa, gsb, ssa, ssb):
        c = lax.axis_index("c")
        s = lax.axis_index("s")
        wid = c * NS + s
        pltpu.sync_copy(src_hbm.at[wid], src_v)
        pltpu.sync_copy(dst_hbm.at[wid], dst_v)

        for y_hbm, out_hbm in ((ya_hbm, outa_hbm), (yb_hbm, outb_hbm)):

            def gat(jc, k, sem):
                pltpu.async_copy(y_hbm.at[src_v.at[jc]], gbuf.at[k], sem)

            def gat_wait(k, sem):
                pltpu.make_async_copy(y_hbm.at[src_v.at[0]], gbuf.at[k],
                                      sem).wait()

            def sca(jc, k, sem):
                pltpu.async_copy(gbuf.at[k], acc_sp.at[dst_v.at[jc]], sem,
                                 add=True)

            def sca_wait(k, sem):
                pltpu.make_async_copy(gbuf.at[k], acc_sp.at[dst_v.at[0]],
                                      sem).wait()

            pltpu.sync_copy(zeros_hbm, gbuf.at[0])
            for off, sz in _row_chunks(ZR, CHUNK):
                pltpu.sync_copy(gbuf.at[0].at[pl.ds(0, sz)],
                                acc_sp.at[pl.ds(s * ZR + off, sz)])
            plsc.subcore_barrier()

            for k in range(NSLOT):
                gat(k, k, gsa)

            def step(i, carry):
                b8 = 2 * NSLOT * i
                for k in range(NSLOT):
                    gat(b8 + NSLOT + k, NSLOT + k, gsb)
                for k in range(NSLOT):
                    gat_wait(k, gsa)
                for k in range(NSLOT):
                    sca(b8 + k, k, ssa)
                for k in range(NSLOT):
                    sca_wait(k, ssa)
                for k in range(NSLOT):
                    gat(jnp.minimum(b8 + 2 * NSLOT + k, CPT - 1), k, gsa)
                for k in range(NSLOT):
                    gat_wait(NSLOT + k, gsb)
                for k in range(NSLOT):
                    sca(b8 + NSLOT + k, NSLOT + k, ssb)
                for k in range(NSLOT):
                    sca_wait(NSLOT + k, ssb)
                return carry

            lax.fori_loop(0, CPT // (2 * NSLOT), step, 0)
            for k in range(NSLOT):
                gat_wait(k, gsa)
            plsc.subcore_barrier()
            for off, sz in _row_chunks(ZR, CHUNK):
                pltpu.sync_copy(acc_sp.at[pl.ds(s * ZR + off, sz)],
                                gbuf.at[0].at[pl.ds(0, sz)])
                pltpu.sync_copy(gbuf.at[0].at[pl.ds(0, sz)],
                                out_hbm.at[c].at[pl.ds(s * ZR + off, sz)])

    return _sc_degree, _sc_msgpass


_RB = 2000
_GRID = N // _RB


def _tc1_body(x_ref, w1_ref, degp_ref, ya_ref, yb_ref, dinv_ref):
    deg = degp_ref[0, :, 0:1] + degp_ref[1, :, 0:1] + 1.0
    dinv = lax.rsqrt(deg)
    xw = jnp.dot(x_ref[...], w1_ref[...], preferred_element_type=jnp.float32)
    y = xw * dinv
    ya_ref[...] = y[:, :HW]
    yb_ref[...] = y[:, HW:]
    dinv_ref[...] = dinv


def _tc1(x, w1, degp):
    return pl.pallas_call(
        _tc1_body,
        grid=(_GRID,),
        in_specs=[
            pl.BlockSpec((_RB, D), lambda i: (i, 0)),
            pl.BlockSpec((D, H), lambda i: (0, 0)),
            pl.BlockSpec((NC, _RB, DEGW), lambda i: (0, i, 0)),
        ],
        out_specs=[
            pl.BlockSpec((_RB, HW), lambda i: (i, 0)),
            pl.BlockSpec((_RB, HW), lambda i: (i, 0)),
            pl.BlockSpec((_RB, 1), lambda i: (i, 0)),
        ],
        out_shape=[
            jax.ShapeDtypeStruct((N, HW), jnp.float32),
            jax.ShapeDtypeStruct((N, HW), jnp.float32),
            jax.ShapeDtypeStruct((N, 1), jnp.float32),
        ],
    )(x, w1, degp)


def _gcn_out(acca_ref, accb_ref, ya_ref, yb_ref, dinv, b_ref):
    ha = acca_ref[0] + acca_ref[1] + ya_ref[...]
    hb = accb_ref[0] + accb_ref[1] + yb_ref[...]
    h = jnp.concatenate([ha, hb], axis=1) * dinv + b_ref[...]
    return jnp.maximum(h, 0.0)


def _tc2_body(acca_ref, accb_ref, ya_ref, yb_ref, dinv_ref, b1_ref, w2_ref,
              y2a_ref, y2b_ref):
    dinv = dinv_ref[...]
    h = _gcn_out(acca_ref, accb_ref, ya_ref, yb_ref, dinv, b1_ref)
    y2 = jnp.dot(h, w2_ref[...], preferred_element_type=jnp.float32) * dinv
    y2a_ref[...] = y2[:, :HW]
    y2b_ref[...] = y2[:, HW:]


def _tc2(acca, accb, ya, yb, dinv, b1, w2):
    return pl.pallas_call(
        _tc2_body,
        grid=(_GRID,),
        in_specs=[
            pl.BlockSpec((NC, _RB, HW), lambda i: (0, i, 0)),
            pl.BlockSpec((NC, _RB, HW), lambda i: (0, i, 0)),
            pl.BlockSpec((_RB, HW), lambda i: (i, 0)),
            pl.BlockSpec((_RB, HW), lambda i: (i, 0)),
            pl.BlockSpec((_RB, 1), lambda i: (i, 0)),
            pl.BlockSpec((1, H), lambda i: (0, 0)),
            pl.BlockSpec((H, H), lambda i: (0, 0)),
        ],
        out_specs=[
            pl.BlockSpec((_RB, HW), lambda i: (i, 0)),
            pl.BlockSpec((_RB, HW), lambda i: (i, 0)),
        ],
        out_shape=[
            jax.ShapeDtypeStruct((N, HW), jnp.float32),
            jax.ShapeDtypeStruct((N, HW), jnp.float32),
        ],
    )(acca, accb, ya, yb, dinv, b1, w2)


def _tc3_body(acca_ref, accb_ref, ya_ref, yb_ref, dinv_ref, b2_ref, batch_ref,
              seq_ref, wf_ref, bf_ref, wcat_ref, bcat_ref, out_ref,
              sums_ref, cnt_ref):
    i = pl.program_id(0)
    g = _gcn_out(acca_ref, accb_ref, ya_ref, yb_ref, dinv_ref[...], b2_ref)
    ids = batch_ref[...]
    cols = lax.broadcasted_iota(jnp.int32, (_RB, B), 1)
    a = (cols == ids).astype(jnp.float32)
    pa = lax.dot_general(a, g, (((0,), (0,)), ((), ())),
                         preferred_element_type=jnp.float32)
    ca = lax.dot_general(a, jnp.ones((_RB, 1), jnp.float32),
                         (((0,), (0,)), ((), ())),
                         preferred_element_type=jnp.float32)

    @pl.when(i == 0)
    def _():
        sums_ref[...] = jnp.zeros_like(sums_ref)
        cnt_ref[...] = jnp.zeros_like(cnt_ref)

    sums_ref[...] += pa
    cnt_ref[...] += ca

    @pl.when(i == _GRID - 1)
    def _():
        pooled = sums_ref[...] / jnp.maximum(cnt_ref[...], 1.0)
        fused = jnp.concatenate([seq_ref[...], pooled], axis=1)
        f2 = jnp.maximum(
            jnp.dot(fused, wf_ref[...], preferred_element_type=jnp.float32)
            + bf_ref[...], 0.0)
        out_ref[...] = (jnp.dot(f2, wcat_ref[...],
                                preferred_element_type=jnp.float32)
                        + bcat_ref[...])


def _tc3(acca, accb, ya, yb, dinv, b2, batch2d, seq_emb, wf, bf, wcat, bcat):
    return pl.pallas_call(
        _tc3_body,
        grid=(_GRID,),
        in_specs=[
            pl.BlockSpec((NC, _RB, HW), lambda i: (0, i, 0)),
            pl.BlockSpec((NC, _RB, HW), lambda i: (0, i, 0)),
            pl.BlockSpec((_RB, HW), lambda i: (i, 0)),
            pl.BlockSpec((_RB, HW), lambda i: (i, 0)),
            pl.BlockSpec((_RB, 1), lambda i: (i, 0)),
            pl.BlockSpec((1, H), lambda i: (0, 0)),
            pl.BlockSpec((_RB, 1), lambda i: (i, 0)),
            pl.BlockSpec((B, D), lambda i: (0, 0)),
            pl.BlockSpec((D + H, 128), lambda i: (0, 0)),
            pl.BlockSpec((1, 128), lambda i: (0, 0)),
            pl.BlockSpec((128, 128), lambda i: (0, 0)),
            pl.BlockSpec((1, 128), lambda i: (0, 0)),
        ],
        out_specs=pl.BlockSpec((B, 128), lambda i: (0, 0)),
        out_shape=jax.ShapeDtypeStruct((B, 128), jnp.float32),
        scratch_shapes=[
            pltpu.VMEM((B, H), jnp.float32),
            pltpu.VMEM((B, 1), jnp.float32),
        ],
    )(acca, accb, ya, yb, dinv, b2, batch2d, seq_emb, wf, bf, wcat, bcat)


def kernel(seq_emb, x, edge_index, batch, W1, b1, W2, b2, Wf, bf, Wtp, btp, Wp, bp):
    pad = ET - E
    src_p = jnp.concatenate(
        [edge_index[0], jnp.zeros((pad,), jnp.int32)]).reshape(NW, CPT, CHUNK)
    dst_p = jnp.concatenate(
        [edge_index[1], jnp.full((pad,), N, jnp.int32)]).reshape(NW, CPT, CHUNK)

    ones_deg = jnp.ones((CHUNK, DEGW), jnp.float32)
    zeros_deg = jnp.zeros((ZR, DEGW), jnp.float32)
    zeros_row = jnp.zeros((CHUNK, HW), jnp.float32)

    sc_degree, sc_msgpass = _sc_kernels()
    degp = sc_degree(dst_p, ones_deg, zeros_deg)

    y1a, y1b, dinv = _tc1(x, W1, degp)
    acc1a, acc1b = sc_msgpass(y1a, y1b, src_p, dst_p, zeros_row)
    y2a, y2b = _tc2(acc1a, acc1b, y1a, y1b, dinv, b1.reshape(1, H), W2)
    acc2a, acc2b = sc_msgpass(y2a, y2b, src_p, dst_p, zeros_row)

    wcat = jnp.zeros((128, 128), jnp.float32)
    wcat = wcat.at[:, 0:2].set(Wtp).at[:, 2:3].set(Wp)
    bcat = jnp.zeros((1, 128), jnp.float32)
    bcat = bcat.at[0, 0:2].set(btp).at[0, 2:3].set(bp)

    out3 = _tc3(acc2a, acc2b, y2a, y2b, dinv, b2.reshape(1, H),
                batch.reshape(N, 1), seq_emb, Wf, bf.reshape(1, 128),
                wcat, bcat)
    return (out3[:, 0:2], out3[:, 2:3])
